# Initial kernel scaffold; baseline (speedup 1.0000x reference)
import jax
import jax.numpy as jnp
from jax.experimental import pallas as pl


def kernel(dense_features, embedding_ids, emb_table, Wb0, bb0, Wb1, bb1, Wb2, bb2, Wt0, bt0, Wt1, bt1, Wt2, bt2, Wt3, bt3):
    raise NotImplementedError("write your pallas kernel here")


# speedup vs baseline: 2.9602x; 2.9602x over previous
import functools

import numpy as np
import jax
import jax.numpy as jnp
from jax import lax
from jax.experimental import pallas as pl
from jax.experimental.pallas import tpu as pltpu
from jax.experimental.pallas import tpu_sc as plsc

BATCH = 4096
NUM_DENSE = 13
EMB = 128
NFIELDS = 26
NF1 = NFIELDS + 1
IDS = BATCH * NFIELDS
INTER = NF1 * NF1

_NC, _NS = 2, 16
_NW = _NC * _NS
_BPW = IDS // _NW
_CHUNK = 128
_NCHUNK = _BPW // _CHUNK


@functools.cache
def _get_sc_gather():
    mesh = plsc.VectorSubcoreMesh(core_axis_name="c", subcore_axis_name="s")

    @functools.partial(
        pl.kernel,
        out_type=jax.ShapeDtypeStruct((IDS, EMB), jnp.float32),
        mesh=mesh,
        scratch_types=[
            pltpu.VMEM((_BPW,), jnp.int32),
            pltpu.VMEM((_CHUNK, EMB), jnp.float32),
            pltpu.SemaphoreType.DMA,
        ],
    )
    def _sc_gather(ids_hbm, table_hbm, out_hbm, idx_v, rows_v, sem):
        wid = lax.axis_index("s") * _NC + lax.axis_index("c")
        base = wid * _BPW
        pltpu.sync_copy(ids_hbm.at[pl.ds(base, _BPW)], idx_v)

        def step(j, carry):
            off = pl.multiple_of(j * _CHUNK, _CHUNK)
            pltpu.async_copy(
                table_hbm.at[idx_v.at[pl.ds(off, _CHUNK)]], rows_v, sem
            ).wait()
            pltpu.sync_copy(rows_v, out_hbm.at[pl.ds(base + off, _CHUNK)])
            return carry

        lax.fori_loop(0, _NCHUNK, step, 0)

    return _sc_gather


_BB = 256


def _tc_body(x_ref, emb_ref, wb0, bb0, wb1, bb1, wb2, bb2,
             w0d, w0f, bt0, wt1, bt1, wt2, bt2, wt3, bt3, out_ref):
    f32 = jnp.float32
    x = x_ref[...]
    d = jnp.maximum(jnp.dot(x, wb0[...], preferred_element_type=f32) + bb0[...], 0.0)
    d = jnp.maximum(jnp.dot(d, wb1[...], preferred_element_type=f32) + bb1[...], 0.0)
    d = jnp.maximum(jnp.dot(d, wb2[...], preferred_element_type=f32) + bb2[...], 0.0)
    c = jnp.concatenate([d, emb_ref[...]], axis=1)
    cb = c.reshape(_BB, NF1, EMB)
    z = lax.dot_general(cb, cb, (((2,), (2,)), ((0,), (0,))),
                        preferred_element_type=f32)
    zv = z.reshape(_BB, INTER)
    h = (jnp.dot(d, w0d[...], preferred_element_type=f32)
         + jnp.dot(zv, w0f[...], preferred_element_type=f32) + bt0[...])
    h = jnp.maximum(h, 0.0)
    h = jnp.maximum(jnp.dot(h, wt1[...], preferred_element_type=f32) + bt1[...], 0.0)
    h = jnp.maximum(jnp.dot(h, wt2[...], preferred_element_type=f32) + bt2[...], 0.0)
    out_ref[...] = jnp.dot(h, wt3[...], preferred_element_type=f32) + bt3[...]


def _full(shape):
    return pl.BlockSpec(shape, lambda i: (0, 0))


def _dense_call(x, emb2, wb0, bb0, wb1, bb1, wb2, bb2,
                w0d, w0f, bt0, wt1, bt1, wt2, bt2, wt3, bt3,
                interpret=False):
    grid = (BATCH // _BB,)
    in_specs = [
        pl.BlockSpec((_BB, NUM_DENSE), lambda i: (i, 0)),
        pl.BlockSpec((_BB, NFIELDS * EMB), lambda i: (i, 0)),
        _full(wb0.shape), _full(bb0.shape),
        _full(wb1.shape), _full(bb1.shape),
        _full(wb2.shape), _full(bb2.shape),
        _full(w0d.shape), _full(w0f.shape), _full(bt0.shape),
        _full(wt1.shape), _full(bt1.shape),
        _full(wt2.shape), _full(bt2.shape),
        _full(wt3.shape), _full(bt3.shape),
    ]
    return pl.pallas_call(
        _tc_body,
        grid=grid,
        in_specs=in_specs,
        out_specs=pl.BlockSpec((_BB, 1), lambda i: (i, 0)),
        out_shape=jax.ShapeDtypeStruct((BATCH, 1), jnp.float32),
        interpret=interpret,
    )(x, emb2, wb0, bb0, wb1, bb1, wb2, bb2,
      w0d, w0f, bt0, wt1, bt1, wt2, bt2, wt3, bt3)


_li, _lj = np.triu_indices(NF1, k=1)
_K = np.zeros((NF1, NF1), dtype=np.int32)
_K[_li, _lj] = np.arange(len(_li), dtype=np.int32)
_K[_lj, _li] = np.arange(len(_li), dtype=np.int32)
_KIDX = _K.reshape(-1)
_KMASK = (~np.eye(NF1, dtype=bool)).reshape(-1, 1).astype(np.float32)


def kernel(dense_features, embedding_ids, emb_table,
           Wb0, bb0, Wb1, bb1, Wb2, bb2,
           Wt0, bt0, Wt1, bt1, Wt2, bt2, Wt3, bt3):
    ids = embedding_ids.reshape(-1).astype(jnp.int32)
    emb_flat = _get_sc_gather()(ids, emb_table)
    emb2 = emb_flat.reshape(BATCH, NFIELDS * EMB)
    w0d = Wt0[:EMB]
    w0f = 0.5 * (Wt0[EMB:][_KIDX] * _KMASK)
    out2 = _dense_call(
        dense_features, emb2,
        Wb0, bb0.reshape(1, -1), Wb1, bb1.reshape(1, -1), Wb2, bb2.reshape(1, -1),
        w0d, w0f, bt0.reshape(1, -1),
        Wt1, bt1.reshape(1, -1), Wt2, bt2.reshape(1, -1), Wt3, bt3.reshape(1, -1))
    return out2.reshape(BATCH)


# speedup vs baseline: 3.6946x; 1.2481x over previous
import functools

import numpy as np
import jax
import jax.numpy as jnp
from jax import lax
from jax.experimental import pallas as pl
from jax.experimental.pallas import tpu as pltpu
from jax.experimental.pallas import tpu_sc as plsc

BATCH = 4096
NUM_DENSE = 13
EMB = 128
NFIELDS = 26
NF1 = NFIELDS + 1
IDS = BATCH * NFIELDS
INTER = NF1 * NF1

_NC, _NS = 2, 16
_NW = _NC * _NS
_BPW = IDS // _NW
_CHUNK = 128
_NCHUNK = _BPW // _CHUNK


@functools.cache
def _get_sc_gather():
    mesh = plsc.VectorSubcoreMesh(core_axis_name="c", subcore_axis_name="s")

    @functools.partial(
        pl.kernel,
        out_type=jax.ShapeDtypeStruct((IDS, EMB), jnp.float32),
        mesh=mesh,
        scratch_types=[
            pltpu.VMEM((_BPW,), jnp.int32),
            pltpu.VMEM((_CHUNK, EMB), jnp.float32),
            pltpu.SemaphoreType.DMA,
        ],
    )
    def _sc_gather(ids_hbm, table_hbm, out_hbm, idx_v, rows_v, sem):
        wid = lax.axis_index("s") * _NC + lax.axis_index("c")
        base = wid * _BPW
        pltpu.sync_copy(ids_hbm.at[pl.ds(base, _BPW)], idx_v)

        def step(j, carry):
            off = pl.multiple_of(j * _CHUNK, _CHUNK)
            pltpu.async_copy(
                table_hbm.at[idx_v.at[pl.ds(off, _CHUNK)]], rows_v, sem
            ).wait()
            pltpu.sync_copy(rows_v, out_hbm.at[pl.ds(base + off, _CHUNK)])
            return carry

        lax.fori_loop(0, _NCHUNK, step, 0)

    return _sc_gather


_BB = 256


def _tc_body(x_ref, emb_ref, wb0, bb0, wb1, bb1, wb2, bb2,
             w0d, wde, wee, bt0, wt1, bt1, wt2, bt2, wt3, bt3, out_ref):
    f32 = jnp.float32
    x = x_ref[...]
    d = jnp.maximum(jnp.dot(x, wb0[...], preferred_element_type=f32) + bb0[...], 0.0)
    d = jnp.maximum(jnp.dot(d, wb1[...], preferred_element_type=f32) + bb1[...], 0.0)
    d = jnp.maximum(jnp.dot(d, wb2[...], preferred_element_type=f32) + bb2[...], 0.0)
    e3 = emb_ref[...].reshape(_BB, NFIELDS, EMB)
    zde = lax.dot_general(d, e3, (((1,), (2,)), ((0,), (0,))),
                          preferred_element_type=f32)
    zee = lax.dot_general(e3, e3, (((2,), (2,)), ((0,), (0,))),
                          preferred_element_type=f32)
    zv = zee.reshape(_BB, NFIELDS * NFIELDS)
    h = (jnp.dot(d, w0d[...], preferred_element_type=f32)
         + jnp.dot(zde, wde[...], preferred_element_type=f32)
         + jnp.dot(zv, wee[...], preferred_element_type=f32)
         + bt0[...])
    h = jnp.maximum(h, 0.0)
    h = jnp.maximum(jnp.dot(h, wt1[...], preferred_element_type=f32) + bt1[...], 0.0)
    h = jnp.maximum(jnp.dot(h, wt2[...], preferred_element_type=f32) + bt2[...], 0.0)
    out_ref[...] = jnp.dot(h, wt3[...], preferred_element_type=f32) + bt3[...]


def _full(shape):
    if len(shape) == 3:
        return pl.BlockSpec(shape, lambda i: (0, 0, 0))
    return pl.BlockSpec(shape, lambda i: (0, 0))


def _dense_call(x, emb_flat, wb0, bb0, wb1, bb1, wb2, bb2,
                w0d, wde, wee, bt0, wt1, bt1, wt2, bt2, wt3, bt3,
                interpret=False):
    grid = (BATCH // _BB,)
    in_specs = [
        pl.BlockSpec((_BB, NUM_DENSE), lambda i: (i, 0)),
        pl.BlockSpec((_BB * NFIELDS, EMB), lambda i: (i, 0)),
        _full(wb0.shape), _full(bb0.shape),
        _full(wb1.shape), _full(bb1.shape),
        _full(wb2.shape), _full(bb2.shape),
        _full(w0d.shape), _full(wde.shape), _full(wee.shape), _full(bt0.shape),
        _full(wt1.shape), _full(bt1.shape),
        _full(wt2.shape), _full(bt2.shape),
        _full(wt3.shape), _full(bt3.shape),
    ]
    return pl.pallas_call(
        _tc_body,
        grid=grid,
        in_specs=in_specs,
        out_specs=pl.BlockSpec((_BB, 1), lambda i: (i, 0)),
        out_shape=jax.ShapeDtypeStruct((BATCH, 1), jnp.float32),
        interpret=interpret,
    )(x, emb_flat, wb0, bb0, wb1, bb1, wb2, bb2,
      w0d, wde, wee, bt0, wt1, bt1, wt2, bt2, wt3, bt3)


_li, _lj = np.triu_indices(NF1, k=1)
_K = np.zeros((NF1, NF1), dtype=np.int32)
_K[_li, _lj] = np.arange(len(_li), dtype=np.int32)
_K[_lj, _li] = np.arange(len(_li), dtype=np.int32)
_K26 = _K[1:, 1:].reshape(-1)
_KMASK26 = (~np.eye(NFIELDS, dtype=bool)).reshape(-1, 1).astype(np.float32)


def kernel(dense_features, embedding_ids, emb_table,
           Wb0, bb0, Wb1, bb1, Wb2, bb2,
           Wt0, bt0, Wt1, bt1, Wt2, bt2, Wt3, bt3):
    ids = embedding_ids.reshape(-1).astype(jnp.int32)
    emb_flat = _get_sc_gather()(ids, emb_table)
    w0d = Wt0[:EMB]
    wde = Wt0[EMB:EMB + NFIELDS]
    wee = 0.5 * (Wt0[EMB:][_K26] * _KMASK26)
    out2 = _dense_call(
        dense_features, emb_flat,
        Wb0, bb0.reshape(1, -1), Wb1, bb1.reshape(1, -1), Wb2, bb2.reshape(1, -1),
        w0d, wde, wee, bt0.reshape(1, -1),
        Wt1, bt1.reshape(1, -1), Wt2, bt2.reshape(1, -1), Wt3, bt3.reshape(1, -1))
    return out2.reshape(BATCH)
